# baseline (speedup 1.0000x reference)
import jax
import jax.numpy as jnp
from jax import lax
from jax.experimental import pallas as pl
from jax.experimental.pallas import tpu as pltpu
from jax.experimental.pallas import tpu_sc as plsc

N = 50000
D = 256
G = 128
SCC = 128
TCC = D - SCC

NC = 2
NS = 16
L = 16
W = NC * NS
SPW = G // W
KD = SCC // L

CH = 128

RB = 200
NBLK = N // RB


def _sc_body(x_hbm, b_hbm, out_hbm, batch_sh, batch_v, buf_v, acc_v, sems):
    cid = lax.axis_index("c")
    sid = lax.axis_index("s")
    wid = sid * NC + cid

    @pl.when(sid == 0)
    def _():
        pltpu.sync_copy(b_hbm, batch_sh)

    plsc.subcore_barrier()
    pltpu.sync_copy(batch_sh, batch_v)

    lanes = lax.iota(jnp.int32, L)
    gtarg = jnp.minimum(wid * SPW + lanes, G)

    def bs_body(_, lohi):
        lo, hi = lohi
        active = lo < hi
        mid = (lo + hi) >> 1
        probe = plsc.load_gather(batch_v, [jnp.minimum(mid, N - 1)])
        pred = probe < gtarg
        lo = jnp.where(active & pred, mid + 1, lo)
        hi = jnp.where(active & jnp.logical_not(pred), mid, hi)
        return lo, hi

    lo, _ = lax.fori_loop(
        0, 17, bs_body,
        (jnp.zeros((L,), jnp.int32), jnp.full((L,), N, jnp.int32)))

    def extract(i):
        return jnp.max(jnp.where(lanes == i, lo, jnp.int32(-1)))

    s0 = extract(0)
    s_end = extract(SPW)

    neg_inf = jnp.full((L,), -jnp.inf, jnp.float32)

    def init_body(k, carry):
        acc_v[pl.ds(pl.multiple_of(L * k, L), L)] = neg_inf
        return carry

    lax.fori_loop(0, SPW * KD, init_body, jnp.int32(0))

    a0 = (s0 >> 3) << 3
    nch = (s_end - a0 + CH - 1) // CH

    def chunk_off(cc):
        return pl.multiple_of(jnp.minimum(a0 + cc * CH, N - CH), 8)

    def slot_base(cc):
        return pl.multiple_of((cc & 1) * CH, 8)

    def issue(cc):
        pltpu.async_copy(x_hbm.at[pl.ds(chunk_off(cc), CH), pl.ds(0, SCC)],
                         buf_v.at[pl.ds(slot_base(cc), CH)],
                         sems.at[cc & 1])

    def wait(cc):
        pltpu.make_async_copy(x_hbm.at[pl.ds(0, CH), pl.ds(0, SCC)],
                              buf_v.at[pl.ds(slot_base(cc), CH)],
                              sems.at[cc & 1]).wait()

    @pl.when(nch > 0)
    def _():
        issue(0)

    def chunk_body(cc, carry):
        wait(cc)

        @pl.when(cc + 1 < nch)
        def _():
            issue(cc + 1)

        base = slot_base(cc)
        off = chunk_off(cc)

        def seg_body(gi, c2):
            b_lo = extract(gi)
            b_hi = extract(gi + 1)
            j_lo = jnp.clip(b_lo - off, 0, CH)
            j_hi = jnp.clip(b_hi - off, 0, CH)
            abase = pl.multiple_of(gi * SCC, L)
            accs = tuple(acc_v[pl.ds(abase + L * k, L)] for k in range(KD))

            def row_body(j, accs):
                return tuple(
                    jnp.maximum(accs[k], buf_v[base + j, pl.ds(L * k, L)])
                    for k in range(KD))

            accs = lax.fori_loop(j_lo, j_hi, row_body, accs)
            for k in range(KD):
                acc_v[pl.ds(abase + L * k, L)] = accs[k]
            return c2

        lax.fori_loop(0, SPW, seg_body, jnp.int32(0))
        return carry

    lax.fori_loop(0, nch, chunk_body, jnp.int32(0))

    pltpu.sync_copy(acc_v, out_hbm.at[pl.ds(wid * SPW * SCC, SPW * SCC)])


def _tc_body(b_ref, x_ref, o_ref):
    i = pl.program_id(0)

    @pl.when(i == 0)
    def _():
        o_ref[...] = jnp.full((G, TCC), -jnp.inf, jnp.float32)

    bv = b_ref[0]
    xblk = x_ref[...]
    g_min = jnp.min(bv)
    g_max = jnp.max(bv)
    riota = lax.broadcasted_iota(jnp.int32, (RB, 1), 0)

    def seg(g, c):
        r_lo = jnp.sum((bv < g).astype(jnp.int32))
        r_hi = jnp.sum((bv <= g).astype(jnp.int32))
        m = (riota >= r_lo) & (riota < r_hi)
        part = jnp.max(jnp.where(m, xblk, -jnp.inf), axis=0, keepdims=True)
        o_ref[pl.ds(g, 1), :] = jnp.maximum(o_ref[pl.ds(g, 1), :], part)
        return c

    lax.fori_loop(g_min, g_max + 1, seg, jnp.int32(0))


@jax.jit
def _hybrid_segment_max(x, batch):
    mesh = plsc.VectorSubcoreMesh(core_axis_name="c", subcore_axis_name="s")
    out_sc = pl.kernel(
        _sc_body,
        out_type=jax.ShapeDtypeStruct((G * SCC,), jnp.float32),
        mesh=mesh,
        compiler_params=pltpu.CompilerParams(needs_layout_passes=False,
                                             use_tc_tiling_on_sc=True),
        scratch_types=[
            pltpu.VMEM_SHARED((N,), jnp.int32),
            pltpu.VMEM((N,), jnp.int32),
            pltpu.VMEM((2 * CH, SCC), jnp.float32),
            pltpu.VMEM((SPW * SCC,), jnp.float32),
            pltpu.SemaphoreType.DMA((2,)),
        ],
    )(x, batch)

    out_tc = pl.pallas_call(
        _tc_body,
        grid=(NBLK,),
        in_specs=[
            pl.BlockSpec((1, 1, RB), lambda i: (i, 0, 0)),
            pl.BlockSpec((RB, TCC), lambda i: (i, 1)),
        ],
        out_specs=pl.BlockSpec((G, TCC), lambda i: (0, 0)),
        out_shape=jax.ShapeDtypeStruct((G, TCC), jnp.float32),
    )(batch.reshape(NBLK, 1, RB), x)

    return jnp.concatenate([out_sc.reshape(G, SCC), out_tc], axis=1)


def kernel(x, batch):
    return _hybrid_segment_max(x, batch)


# speedup vs baseline: 3.9183x; 3.9183x over previous
import jax
import jax.numpy as jnp
from jax import lax
from jax.experimental import pallas as pl
from jax.experimental.pallas import tpu as pltpu
from jax.experimental.pallas import tpu_sc as plsc

N = 50000
D = 256
G = 128

NC = 2
NS = 16
L = 16
W = NC * NS
SPW = G // W
KD = D // L

CH = 128


def _sc_body(x_hbm, b_hbm, out_hbm, batch_sh, batch_v, buf_v, acc_v, sems):
    cid = lax.axis_index("c")
    sid = lax.axis_index("s")
    wid = sid * NC + cid

    @pl.when(sid == 0)
    def _():
        pltpu.sync_copy(b_hbm, batch_sh)

    plsc.subcore_barrier()
    pltpu.sync_copy(batch_sh, batch_v)

    lanes = lax.iota(jnp.int32, L)
    gtarg = jnp.minimum(wid * SPW + lanes, G)

    def bs_body(_, lohi):
        lo, hi = lohi
        active = lo < hi
        mid = (lo + hi) >> 1
        probe = plsc.load_gather(batch_v, [jnp.minimum(mid, N - 1)])
        pred = probe < gtarg
        lo = jnp.where(active & pred, mid + 1, lo)
        hi = jnp.where(active & jnp.logical_not(pred), mid, hi)
        return lo, hi

    lo, _ = lax.fori_loop(
        0, 17, bs_body,
        (jnp.zeros((L,), jnp.int32), jnp.full((L,), N, jnp.int32)))

    def extract(i):
        return jnp.max(jnp.where(lanes == i, lo, jnp.int32(-1)))

    s0 = extract(0)
    s_end = extract(SPW)

    neg_inf = jnp.full((L,), -jnp.inf, jnp.float32)

    def init_body(k, carry):
        acc_v[pl.ds(pl.multiple_of(L * k, L), L)] = neg_inf
        return carry

    lax.fori_loop(0, SPW * KD, init_body, jnp.int32(0))

    a0 = (s0 >> 3) << 3
    nch = (s_end - a0 + CH - 1) // CH

    def chunk_off(cc):
        return pl.multiple_of(jnp.minimum(a0 + cc * CH, N - CH), 8)

    def slot_base(cc):
        return pl.multiple_of((cc & 1) * CH, 8)

    def issue(cc):
        pltpu.async_copy(x_hbm.at[pl.ds(chunk_off(cc), CH)],
                         buf_v.at[pl.ds(slot_base(cc), CH)],
                         sems.at[cc & 1])

    def wait(cc):
        pltpu.make_async_copy(x_hbm.at[pl.ds(0, CH)],
                              buf_v.at[pl.ds(slot_base(cc), CH)],
                              sems.at[cc & 1]).wait()

    @pl.when(nch > 0)
    def _():
        issue(0)

    def chunk_body(cc, carry):
        wait(cc)

        @pl.when(cc + 1 < nch)
        def _():
            issue(cc + 1)

        base = slot_base(cc)
        off = chunk_off(cc)

        def seg_body(gi, c2):
            b_lo = extract(gi)
            b_hi = extract(gi + 1)
            j_lo = jnp.clip(b_lo - off, 0, CH)
            j_hi = jnp.clip(b_hi - off, 0, CH)
            abase = pl.multiple_of(gi * D, L)
            accs = tuple(acc_v[pl.ds(abase + L * k, L)] for k in range(KD))

            def row_body(j, accs):
                return tuple(
                    jnp.maximum(accs[k], buf_v[base + j, pl.ds(L * k, L)])
                    for k in range(KD))

            accs = lax.fori_loop(j_lo, j_hi, row_body, accs)
            for k in range(KD):
                acc_v[pl.ds(abase + L * k, L)] = accs[k]
            return c2

        lax.fori_loop(0, SPW, seg_body, jnp.int32(0))
        return carry

    lax.fori_loop(0, nch, chunk_body, jnp.int32(0))

    pltpu.sync_copy(acc_v, out_hbm.at[pl.ds(wid * SPW * D, SPW * D)])


@jax.jit
def _sc_segment_max(x, batch):
    mesh = plsc.VectorSubcoreMesh(core_axis_name="c", subcore_axis_name="s")
    return pl.kernel(
        _sc_body,
        out_type=jax.ShapeDtypeStruct((G * D,), jnp.float32),
        mesh=mesh,
        compiler_params=pltpu.CompilerParams(needs_layout_passes=False,
                                             use_tc_tiling_on_sc=True),
        scratch_types=[
            pltpu.VMEM_SHARED((N,), jnp.int32),
            pltpu.VMEM((N,), jnp.int32),
            pltpu.VMEM((2 * CH, D), jnp.float32),
            pltpu.VMEM((SPW * D,), jnp.float32),
            pltpu.SemaphoreType.DMA((2,)),
        ],
    )(x, batch)


def kernel(x, batch):
    out = _sc_segment_max(x, batch)
    return out.reshape(G, D)
